# baseline (speedup 1.0000x reference)
import functools

import jax
import jax.numpy as jnp
from jax import lax
from jax.experimental import pallas as pl
from jax.experimental.pallas import tpu as pltpu
from jax.experimental.pallas import tpu_sc as plsc

_VOCAB = 1000000
_EMB = 64
_B = 4096
_L = 200
_SCALE = 8.0

_NC = 2
_NS = 16
_NW = _NC * _NS

_NBLK = (_VOCAB + 127) // 128
_SCR_ROWS = (_NBLK * 128) // 2
_BLK_LO = _NBLK // _NW
_BLK_EXTRA = _NBLK - _BLK_LO * _NW


def _transpose_table(table_t):
    mesh = plsc.VectorSubcoreMesh(core_axis_name="c", subcore_axis_name="s")

    @functools.partial(
        pl.kernel,
        mesh=mesh,
        compiler_params=pltpu.CompilerParams(needs_layout_passes=False),
        out_type=jax.ShapeDtypeStruct((_SCR_ROWS, 128), jnp.float32),
        scratch_types=[
            pltpu.VMEM((2, _EMB, 128), jnp.float32),
            pltpu.VMEM((2, _EMB, 128), jnp.float32),
            pltpu.SemaphoreType.DMA,
            pltpu.SemaphoreType.DMA,
        ],
    )
    def ka(tab_hbm, scr_hbm, vin, vout, semi0, semi1):
        wid = lax.axis_index("s") * _NC + lax.axis_index("c")
        nblk = jnp.where(wid < _BLK_EXTRA, _BLK_LO + 1, _BLK_LO)
        blk0 = wid * _BLK_LO + jnp.minimum(wid, _BLK_EXTRA)

        lane = lax.iota(jnp.int32, 16)
        vvecs = [16 * g + lane for g in range(8)]
        vvecs64 = [(16 * g + lane) * _EMB for g in range(8)]

        def vsrc(i):
            v0 = pl.multiple_of((blk0 + i) * 128, 128)
            return tab_hbm.at[:, pl.ds(v0, 128)]

        def fire(i, buf, sem):
            return pltpu.async_copy(vsrc(i), vin.at[buf], sem)

        def transpose_block(i, buf):
            for g in range(8):
                vvec = vvecs[g]
                vvec64 = vvecs64[g]

                def e_body(e0, carry):
                    evec = (e0 + lane) & (_EMB - 1)
                    x = plsc.load_gather(vin.at[buf], [evec, vvec]) * _SCALE
                    flat = vvec64 + evec
                    plsc.store_scatter(
                        vout.at[buf], [flat >> 7, flat & 127], x
                    )
                    return carry

                lax.fori_loop(0, _EMB, e_body, 0, unroll=16)
            p0 = pl.multiple_of((blk0 + i) * 64, 64)
            pltpu.sync_copy(vout.at[buf], scr_hbm.at[pl.ds(p0, 64)])

        fire(0, 0, semi0)

        def pair(j, carry):
            i0 = 2 * j

            @pl.when(i0 + 1 < nblk)
            def _():
                fire(i0 + 1, 1, semi1)

            @pl.when(i0 < nblk)
            def _():
                pltpu.make_async_copy(vsrc(i0), vin.at[0], semi0).wait()
                transpose_block(i0, 0)

            @pl.when(i0 + 2 < nblk)
            def _():
                fire(i0 + 2, 0, semi0)

            @pl.when(i0 + 1 < nblk)
            def _():
                pltpu.make_async_copy(vsrc(i0 + 1), vin.at[1], semi1).wait()
                transpose_block(i0 + 1, 1)

            return carry

        lax.fori_loop(0, (_BLK_LO + 2) // 2, pair, 0)

    return ka(table_t)


def _gather_rows(tokens_t, scr):
    mesh = plsc.VectorSubcoreMesh(core_axis_name="c", subcore_axis_name="s")

    @functools.partial(
        pl.kernel,
        mesh=mesh,
        compiler_params=pltpu.CompilerParams(needs_layout_passes=False),
        out_type=jax.ShapeDtypeStruct((_L, 8, _NW, 8, 128), jnp.float32),
        scratch_types=[
            pltpu.VMEM((2, 128), jnp.int32),
            pltpu.VMEM((2, 128), jnp.int32),
            pltpu.VMEM((2, 128), jnp.int32),
            pltpu.VMEM((2, 128, 128), jnp.float32),
            pltpu.VMEM((2, 8, 8, 128), jnp.float32),
            pltpu.SemaphoreType.DMA,
            pltpu.SemaphoreType.DMA,
            pltpu.SemaphoreType.DMA,
            pltpu.SemaphoreType.DMA,
        ],
    )
    def kb(tok_hbm, scr_hbm, out_hbm, idx_v, idxp_v, hv_v, rows_v, ob_v,
           semg0, semg1, semw0, semw1):
        wid = lax.axis_index("s") * _NC + lax.axis_index("c")
        b0 = pl.multiple_of(wid * 128, 128)
        lane = lax.iota(jnp.int32, 16)
        bvecs = [16 * g + lane for g in range(8)]

        def fire(l, buf, sem):
            pltpu.sync_copy(tok_hbm.at[l, pl.ds(b0, 128)], idx_v.at[buf])
            for g in range(8):
                sl = pl.ds(16 * g, 16)
                idxp_v[buf, sl] = idx_v[buf, sl] >> 1
            return pltpu.async_copy(
                scr_hbm.at[idxp_v.at[buf]], rows_v.at[buf], sem
            )

        def wait_gather(buf, sem):
            pltpu.make_async_copy(
                scr_hbm.at[idxp_v.at[buf]], rows_v.at[buf], sem
            ).wait()

        def owin(l):
            return out_hbm.at[l, :, wid, :, :]

        def extract(l, buf, semw):
            for g in range(8):
                sl = pl.ds(16 * g, 16)
                hv_v[buf, sl] = (idx_v[buf, sl] & 1) * _EMB
            ob2d = ob_v.at[buf].reshape(_EMB, 128)
            for g in range(8):
                bvec = bvecs[g]
                hvec = plsc.load_gather(hv_v.at[buf], [bvec])

                def e_body(e0, carry):
                    evec = (e0 + lane) & (_EMB - 1)
                    x = plsc.load_gather(rows_v.at[buf], [bvec, hvec + evec])
                    plsc.store_scatter(ob2d, [evec, bvec], x)
                    return carry

                lax.fori_loop(0, _EMB, e_body, 0, unroll=16)
            return pltpu.async_copy(ob_v.at[buf], owin(l), semw)

        def drain_write(l, buf, semw):
            pltpu.make_async_copy(ob_v.at[buf], owin(l), semw).wait()

        fire(0, 0, semg0)

        def pair(j, carry):
            l0 = 2 * j
            fire(l0 + 1, 1, semg1)
            wait_gather(0, semg0)

            @pl.when(l0 >= 2)
            def _():
                drain_write(l0 - 2, 0, semw0)

            extract(l0, 0, semw0)

            @pl.when(l0 + 2 < _L)
            def _():
                fire(l0 + 2, 0, semg0)

            wait_gather(1, semg1)

            @pl.when(l0 >= 2)
            def _():
                drain_write(l0 - 1, 1, semw1)

            extract(l0 + 1, 1, semw1)
            return carry

        lax.fori_loop(0, _L // 2, pair, 0)
        drain_write(_L - 2, 0, semw0)
        drain_write(_L - 1, 1, semw1)

    return kb(tokens_t, scr)


def kernel(tokens, table):
    table_t = table.T
    tokens_t = tokens.T.astype(jnp.int32)
    scr = _transpose_table(table_t)
    out5 = _gather_rows(tokens_t, scr)
    return out5.transpose(2, 4, 0, 1, 3).reshape(_B, _L, _EMB)


# speedup vs baseline: 1.4224x; 1.4224x over previous
import functools

import jax
import jax.numpy as jnp
from jax import lax
from jax.experimental import pallas as pl
from jax.experimental.pallas import tpu as pltpu
from jax.experimental.pallas import tpu_sc as plsc

_VOCAB = 1000000
_EMB = 64
_B = 4096
_L = 200
_SCALE = 8.0

_NC = 2
_NS = 16
_NW = _NC * _NS

_NBLK = (_VOCAB + 127) // 128
_SCR_ROWS = (_NBLK * 128) // 2
_BLK_LO = _NBLK // _NW
_BLK_EXTRA = _NBLK - _BLK_LO * _NW


def _transpose_table(table_t):
    mesh = plsc.VectorSubcoreMesh(core_axis_name="c", subcore_axis_name="s")

    @functools.partial(
        pl.kernel,
        mesh=mesh,
        compiler_params=pltpu.CompilerParams(needs_layout_passes=False),
        out_type=jax.ShapeDtypeStruct((_SCR_ROWS, 128), jnp.float32),
        scratch_types=[
            pltpu.VMEM((2, _EMB, 128), jnp.float32),
            pltpu.VMEM((2, _EMB, 128), jnp.float32),
            pltpu.SemaphoreType.DMA,
            pltpu.SemaphoreType.DMA,
            pltpu.SemaphoreType.DMA,
            pltpu.SemaphoreType.DMA,
        ],
    )
    def ka(tab_hbm, scr_hbm, vin, vout, semi0, semi1, semo0, semo1):
        wid = lax.axis_index("s") * _NC + lax.axis_index("c")
        nblk = jnp.where(wid < _BLK_EXTRA, _BLK_LO + 1, _BLK_LO)
        blk0 = wid * _BLK_LO + jnp.minimum(wid, _BLK_EXTRA)

        lane = lax.iota(jnp.int32, 16)
        evecs = [16 * je + lane for je in range(4)]

        def vsrc(i):
            v0 = pl.multiple_of((blk0 + i) * 128, 128)
            return tab_hbm.at[:, pl.ds(v0, 128)]

        def odst(i):
            p0 = pl.multiple_of((blk0 + i) * 64, 64)
            return scr_hbm.at[pl.ds(p0, 64)]

        def fire_in(i, buf, sem):
            return pltpu.async_copy(vsrc(i), vin.at[buf], sem)

        def wait_in(i, buf, sem):
            pltpu.make_async_copy(vsrc(i), vin.at[buf], sem).wait()

        def drain_out(buf, sem):
            pltpu.make_async_copy(
                vout.at[buf], scr_hbm.at[pl.ds(0, 64)], sem
            ).wait()

        def transpose_block(buf):
            for g in range(8):
                v0g = 16 * g

                def k_body(k, carry):
                    vvec = v0g + ((lane + k) & 15)
                    vvec64 = vvec * _EMB
                    for je in range(4):
                        evec = evecs[je]
                        x = plsc.load_gather(vin.at[buf], [evec, vvec])
                        flat = vvec64 + evec
                        plsc.store_scatter(
                            vout.at[buf],
                            [flat >> 7, flat & 127],
                            x * _SCALE,
                        )
                    return carry

                lax.fori_loop(0, 16, k_body, 0)

        fire_in(0, 0, semi0)

        def pair(j, carry):
            i0 = 2 * j

            @pl.when(i0 + 1 < nblk)
            def _():
                fire_in(i0 + 1, 1, semi1)

            @pl.when(i0 < nblk)
            def _():
                wait_in(i0, 0, semi0)

                @pl.when(i0 >= 2)
                def _():
                    drain_out(0, semo0)

                transpose_block(0)
                pltpu.async_copy(vout.at[0], odst(i0), semo0)

            @pl.when(i0 + 2 < nblk)
            def _():
                fire_in(i0 + 2, 0, semi0)

            @pl.when(i0 + 1 < nblk)
            def _():
                wait_in(i0 + 1, 1, semi1)

                @pl.when(i0 >= 2)
                def _():
                    drain_out(1, semo1)

                transpose_block(1)
                pltpu.async_copy(vout.at[1], odst(i0 + 1), semo1)

            return carry

        lax.fori_loop(0, (_BLK_LO + 2) // 2, pair, 0)
        drain_out(0, semo0)
        drain_out(1, semo1)

    return ka(table_t)


def _gather_rows(tokens_t, scr):
    mesh = plsc.VectorSubcoreMesh(core_axis_name="c", subcore_axis_name="s")

    @functools.partial(
        pl.kernel,
        mesh=mesh,
        compiler_params=pltpu.CompilerParams(needs_layout_passes=False),
        out_type=jax.ShapeDtypeStruct((_L, 8, _NW, 8, 128), jnp.float32),
        scratch_types=[
            pltpu.VMEM((2, 128), jnp.int32),
            pltpu.VMEM((2, 128), jnp.int32),
            pltpu.VMEM((2, 128), jnp.int32),
            pltpu.VMEM((2, 128, 128), jnp.float32),
            pltpu.VMEM((2, 8, 8, 128), jnp.float32),
            pltpu.SemaphoreType.DMA,
            pltpu.SemaphoreType.DMA,
            pltpu.SemaphoreType.DMA,
            pltpu.SemaphoreType.DMA,
        ],
    )
    def kb(tok_hbm, scr_hbm, out_hbm, idx_v, idxp_v, hv_v, rows_v, ob_v,
           semg0, semg1, semw0, semw1):
        wid = lax.axis_index("s") * _NC + lax.axis_index("c")
        b0 = pl.multiple_of(wid * 128, 128)
        lane = lax.iota(jnp.int32, 16)
        evecs = [16 * je + lane for je in range(4)]

        def fire(l, buf, sem):
            pltpu.sync_copy(tok_hbm.at[l, pl.ds(b0, 128)], idx_v.at[buf])
            for g in range(8):
                sl = pl.ds(16 * g, 16)
                idxp_v[buf, sl] = idx_v[buf, sl] >> 1
            return pltpu.async_copy(
                scr_hbm.at[idxp_v.at[buf]], rows_v.at[buf], sem
            )

        def wait_gather(buf, sem):
            pltpu.make_async_copy(
                scr_hbm.at[idxp_v.at[buf]], rows_v.at[buf], sem
            ).wait()

        def owin(l):
            return out_hbm.at[l, :, wid, :, :]

        def extract(l, buf, semw):
            for g in range(8):
                sl = pl.ds(16 * g, 16)
                hv_v[buf, sl] = (idx_v[buf, sl] & 1) * _EMB
            ob2d = ob_v.at[buf].reshape(_EMB, 128)
            for g in range(8):
                b0g = 16 * g

                def k_body(k, carry):
                    bvec = b0g + ((lane + k) & 15)
                    hvec = plsc.load_gather(hv_v.at[buf], [bvec])
                    for je in range(4):
                        evec = evecs[je]
                        x = plsc.load_gather(
                            rows_v.at[buf], [bvec, hvec + evec]
                        )
                        plsc.store_scatter(ob2d, [evec, bvec], x)
                    return carry

                lax.fori_loop(0, 16, k_body, 0)
            return pltpu.async_copy(ob_v.at[buf], owin(l), semw)

        def drain_write(l, buf, semw):
            pltpu.make_async_copy(ob_v.at[buf], owin(l), semw).wait()

        fire(0, 0, semg0)

        def pair(j, carry):
            l0 = 2 * j
            fire(l0 + 1, 1, semg1)
            wait_gather(0, semg0)

            @pl.when(l0 >= 2)
            def _():
                drain_write(l0 - 2, 0, semw0)

            extract(l0, 0, semw0)

            @pl.when(l0 + 2 < _L)
            def _():
                fire(l0 + 2, 0, semg0)

            wait_gather(1, semg1)

            @pl.when(l0 >= 2)
            def _():
                drain_write(l0 - 1, 1, semw1)

            extract(l0 + 1, 1, semw1)
            return carry

        lax.fori_loop(0, _L // 2, pair, 0)
        drain_write(_L - 2, 0, semw0)
        drain_write(_L - 1, 1, semw1)

    return kb(tokens_t, scr)


def kernel(tokens, table):
    table_t = table.T
    tokens_t = tokens.T.astype(jnp.int32)
    scr = _transpose_table(table_t)
    out5 = _gather_rows(tokens_t, scr)
    return out5.transpose(2, 4, 0, 1, 3).reshape(_B, _L, _EMB)


# speedup vs baseline: 1.6289x; 1.1452x over previous
import functools

import jax
import jax.numpy as jnp
from jax import lax
from jax.experimental import pallas as pl
from jax.experimental.pallas import tpu as pltpu
from jax.experimental.pallas import tpu_sc as plsc

_VOCAB = 1000000
_EMB = 64
_B = 4096
_L = 200
_SCALE = 8.0

_NC = 2
_NS = 16
_NW = _NC * _NS

_NBLK = (_VOCAB + 127) // 128
_SCR_ROWS = (_NBLK * 128) // 2
_BLK_LO = _NBLK // _NW
_BLK_EXTRA = _NBLK - _BLK_LO * _NW


def _transpose_table(table_t):
    mesh = plsc.VectorSubcoreMesh(core_axis_name="c", subcore_axis_name="s")

    @functools.partial(
        pl.kernel,
        mesh=mesh,
        compiler_params=pltpu.CompilerParams(needs_layout_passes=False),
        out_type=jax.ShapeDtypeStruct((_SCR_ROWS, 128), jnp.float32),
        scratch_types=[
            pltpu.VMEM((2, _EMB, 128), jnp.float32),
            pltpu.VMEM((2, _EMB, 128), jnp.float32),
            pltpu.SemaphoreType.DMA,
            pltpu.SemaphoreType.DMA,
            pltpu.SemaphoreType.DMA,
            pltpu.SemaphoreType.DMA,
        ],
    )
    def ka(tab_hbm, scr_hbm, vin, vout, semi0, semi1, semo0, semo1):
        wid = lax.axis_index("s") * _NC + lax.axis_index("c")
        nblk = jnp.where(wid < _BLK_EXTRA, _BLK_LO + 1, _BLK_LO)
        blk0 = wid * _BLK_LO + jnp.minimum(wid, _BLK_EXTRA)

        lane = lax.iota(jnp.int32, 16)
        evecs = [16 * je + lane for je in range(4)]

        def vsrc(i):
            v0 = pl.multiple_of((blk0 + i) * 128, 128)
            return tab_hbm.at[:, pl.ds(v0, 128)]

        def odst(i):
            p0 = pl.multiple_of((blk0 + i) * 64, 64)
            return scr_hbm.at[pl.ds(p0, 64)]

        def fire_in(i, buf, sem):
            return pltpu.async_copy(vsrc(i), vin.at[buf], sem)

        def wait_in(i, buf, sem):
            pltpu.make_async_copy(vsrc(i), vin.at[buf], sem).wait()

        def drain_out(buf, sem):
            pltpu.make_async_copy(
                vout.at[buf], scr_hbm.at[pl.ds(0, 64)], sem
            ).wait()

        def transpose_block(buf):
            for g in range(8):
                v0g = 16 * g

                def k_body(k, carry):
                    vvec = v0g + ((lane + k) & 15)
                    vvec64 = vvec * _EMB
                    for je in range(4):
                        evec = evecs[je]
                        x = plsc.load_gather(vin.at[buf], [evec, vvec])
                        flat = vvec64 + evec
                        plsc.store_scatter(
                            vout.at[buf],
                            [flat >> 7, flat & 127],
                            x * _SCALE,
                        )
                    return carry

                lax.fori_loop(0, 16, k_body, 0)

        fire_in(0, 0, semi0)

        def pair(j, carry):
            i0 = 2 * j

            @pl.when(i0 + 1 < nblk)
            def _():
                fire_in(i0 + 1, 1, semi1)

            @pl.when(i0 < nblk)
            def _():
                wait_in(i0, 0, semi0)

                @pl.when(i0 >= 2)
                def _():
                    drain_out(0, semo0)

                transpose_block(0)
                pltpu.async_copy(vout.at[0], odst(i0), semo0)

            @pl.when(i0 + 2 < nblk)
            def _():
                fire_in(i0 + 2, 0, semi0)

            @pl.when(i0 + 1 < nblk)
            def _():
                wait_in(i0 + 1, 1, semi1)

                @pl.when(i0 >= 2)
                def _():
                    drain_out(1, semo1)

                transpose_block(1)
                pltpu.async_copy(vout.at[1], odst(i0 + 1), semo1)

            return carry

        lax.fori_loop(0, (_BLK_LO + 2) // 2, pair, 0)
        drain_out(0, semo0)
        drain_out(1, semo1)

    return ka(table_t)


def _gather_rows(tokens_t, scr):
    mesh = plsc.VectorSubcoreMesh(core_axis_name="c", subcore_axis_name="s")

    @functools.partial(
        pl.kernel,
        mesh=mesh,
        compiler_params=pltpu.CompilerParams(
            needs_layout_passes=False, use_tc_tiling_on_sc=False
        ),
        out_type=jax.ShapeDtypeStruct((_L, 8, _NW, 8, 128), jnp.float32),
        scratch_types=[
            pltpu.VMEM((2, 128), jnp.int32),
            pltpu.VMEM((2, 128, _EMB), jnp.float32),
            pltpu.VMEM((2, 8, 8, 128), jnp.float32),
            pltpu.SemaphoreType.DMA,
            pltpu.SemaphoreType.DMA,
            pltpu.SemaphoreType.DMA,
            pltpu.SemaphoreType.DMA,
        ],
    )
    def kb(tok_hbm, scr_hbm, out_hbm, idx_v, rows_v, ob_v,
           semg0, semg1, semw0, semw1):
        wid = lax.axis_index("s") * _NC + lax.axis_index("c")
        b0 = pl.multiple_of(wid * 128, 128)
        lane = lax.iota(jnp.int32, 16)
        evecs = [16 * je + lane for je in range(4)]

        def fire(l, buf, sem):
            pltpu.sync_copy(tok_hbm.at[l, pl.ds(b0, 128)], idx_v.at[buf])
            return pltpu.async_copy(
                scr_hbm.at[idx_v.at[buf]], rows_v.at[buf], sem
            )

        def wait_gather(buf, sem):
            pltpu.make_async_copy(
                scr_hbm.at[idx_v.at[buf]], rows_v.at[buf], sem
            ).wait()

        def owin(l):
            return out_hbm.at[l, :, wid, :, :]

        def extract(l, buf, semw):
            for g in range(8):
                b0g = 16 * g

                def k_body(k, carry):
                    bvec = b0g + ((lane + k) & 15)
                    for je in range(4):
                        evec = evecs[je]
                        x = plsc.load_gather(rows_v.at[buf], [bvec, evec])
                        plsc.store_scatter(
                            ob_v.at[buf], [evec >> 3, evec & 7, bvec], x
                        )
                    return carry

                lax.fori_loop(0, 16, k_body, 0)
            return pltpu.async_copy(ob_v.at[buf], owin(l), semw)

        def drain_write(l, buf, semw):
            pltpu.make_async_copy(ob_v.at[buf], owin(l), semw).wait()

        fire(0, 0, semg0)

        def pair(j, carry):
            l0 = 2 * j
            fire(l0 + 1, 1, semg1)
            wait_gather(0, semg0)

            @pl.when(l0 >= 2)
            def _():
                drain_write(l0 - 2, 0, semw0)

            extract(l0, 0, semw0)

            @pl.when(l0 + 2 < _L)
            def _():
                fire(l0 + 2, 0, semg0)

            wait_gather(1, semg1)

            @pl.when(l0 >= 2)
            def _():
                drain_write(l0 - 1, 1, semw1)

            extract(l0 + 1, 1, semw1)
            return carry

        lax.fori_loop(0, _L // 2, pair, 0)
        drain_write(_L - 2, 0, semw0)
        drain_write(_L - 1, 1, semw1)

    return kb(tokens_t, scr)


def kernel(tokens, table):
    table_t = table.T
    tokens_t = tokens.T.astype(jnp.int32)
    scr = _transpose_table(table_t)
    scr_rows = scr.reshape(2 * _SCR_ROWS, _EMB)
    out5 = _gather_rows(tokens_t, scr_rows)
    return out5.transpose(2, 4, 0, 1, 3).reshape(_B, _L, _EMB)


# speedup vs baseline: 1.7149x; 1.0528x over previous
import functools

import jax
import jax.numpy as jnp
from jax import lax
from jax.experimental import pallas as pl
from jax.experimental.pallas import tpu as pltpu
from jax.experimental.pallas import tpu_sc as plsc

_VOCAB = 1000000
_EMB = 64
_B = 4096
_L = 200
_SCALE = 8.0

_NC = 2
_NS = 16
_NW = _NC * _NS

_W = 512
_NBLK = (_VOCAB + 127) // 128
_SCR_ROWS = (_NBLK * 128) // 2
_NBIG = (_NBLK * 128 + _W - 1) // _W
_V_LAST = _NBLK * 128 - _W
_BIG_LO = _NBIG // _NW
_BIG_EXTRA = _NBIG - _BIG_LO * _NW


def _transpose_table(table_t):
    mesh = plsc.VectorSubcoreMesh(core_axis_name="c", subcore_axis_name="s")

    @functools.partial(
        pl.kernel,
        mesh=mesh,
        compiler_params=pltpu.CompilerParams(needs_layout_passes=False),
        out_type=jax.ShapeDtypeStruct((_SCR_ROWS, 128), jnp.float32),
        scratch_types=[
            pltpu.VMEM((2, _EMB, _W), jnp.float32),
            pltpu.VMEM((4, _EMB, 128), jnp.float32),
            pltpu.SemaphoreType.DMA,
            pltpu.SemaphoreType.DMA,
            pltpu.SemaphoreType.DMA,
        ],
    )
    def ka(tab_hbm, scr_hbm, vin, vout, semi0, semi1, semo):
        wid = lax.axis_index("s") * _NC + lax.axis_index("c")
        nblk = jnp.where(wid < _BIG_EXTRA, _BIG_LO + 1, _BIG_LO)
        blk0 = wid * _BIG_LO + jnp.minimum(wid, _BIG_EXTRA)

        lane = lax.iota(jnp.int32, 16)
        evecs = [16 * je + lane for je in range(4)]

        def v0_of(i):
            return pl.multiple_of(
                jnp.minimum((blk0 + i) * _W, _V_LAST), 128
            )

        def vsrc(i):
            return tab_hbm.at[:, pl.ds(v0_of(i), _W)]

        def fire_in(i, buf, sem):
            return pltpu.async_copy(vsrc(i), vin.at[buf], sem)

        def wait_in(i, buf, sem):
            pltpu.make_async_copy(vsrc(i), vin.at[buf], sem).wait()

        def drain_out():
            pltpu.make_async_copy(
                vout.at[0], scr_hbm.at[pl.ds(0, 64)], semo
            ).wait()

        def transpose_block(i, buf, first):
            for k4 in range(4):
                @pl.when(jnp.logical_not(first))
                def _():
                    drain_out()

                for g in range(8):
                    v0g = 128 * k4 + 16 * g

                    def k_body(k, carry):
                        vloc = 16 * g + ((lane + k) & 15)
                        vvec = 128 * k4 + vloc
                        vvec64 = vloc * _EMB
                        for je in range(4):
                            evec = evecs[je]
                            x = plsc.load_gather(vin.at[buf], [evec, vvec])
                            flat = vvec64 + evec
                            plsc.store_scatter(
                                vout.at[k4],
                                [flat >> 7, flat & 127],
                                x * _SCALE,
                            )
                        return carry

                    lax.fori_loop(0, 16, k_body, 0)
                p0 = pl.multiple_of(v0_of(i) // 2 + 64 * k4, 64)
                pltpu.async_copy(
                    vout.at[k4], scr_hbm.at[pl.ds(p0, 64)], semo
                )

        fire_in(0, 0, semi0)

        def pair(j, carry):
            i0 = 2 * j

            @pl.when(i0 + 1 < nblk)
            def _():
                fire_in(i0 + 1, 1, semi1)

            @pl.when(i0 < nblk)
            def _():
                wait_in(i0, 0, semi0)
                transpose_block(i0, 0, i0 == 0)

            @pl.when(i0 + 2 < nblk)
            def _():
                fire_in(i0 + 2, 0, semi0)

            @pl.when(i0 + 1 < nblk)
            def _():
                wait_in(i0 + 1, 1, semi1)
                transpose_block(i0 + 1, 1, False)

            return carry

        lax.fori_loop(0, (_BIG_LO + 2) // 2, pair, 0)
        for _ in range(4):
            drain_out()

    return ka(table_t)


def _gather_rows(tokens_t, scr):
    mesh = plsc.VectorSubcoreMesh(core_axis_name="c", subcore_axis_name="s")

    @functools.partial(
        pl.kernel,
        mesh=mesh,
        compiler_params=pltpu.CompilerParams(
            needs_layout_passes=False, use_tc_tiling_on_sc=False
        ),
        out_type=jax.ShapeDtypeStruct((_L, 8, _NW, 8, 128), jnp.float32),
        scratch_types=[
            pltpu.VMEM((_L, 128), jnp.int32),
            pltpu.VMEM((2, 128, _EMB), jnp.float32),
            pltpu.VMEM((2, 8, 8, 128), jnp.float32),
            pltpu.SemaphoreType.DMA,
            pltpu.SemaphoreType.DMA,
            pltpu.SemaphoreType.DMA,
            pltpu.SemaphoreType.DMA,
        ],
    )
    def kb(tok_hbm, scr_hbm, out_hbm, tok_v, rows_v, ob_v,
           semg0, semg1, semw0, semw1):
        wid = lax.axis_index("s") * _NC + lax.axis_index("c")
        b0 = pl.multiple_of(wid * 128, 128)
        lane = lax.iota(jnp.int32, 16)
        evecs = [16 * je + lane for je in range(4)]

        pltpu.sync_copy(tok_hbm.at[:, pl.ds(b0, 128)], tok_v)

        def fire(l, buf, sem):
            return pltpu.async_copy(
                scr_hbm.at[tok_v.at[l]], rows_v.at[buf], sem
            )

        def wait_gather(l, buf, sem):
            pltpu.make_async_copy(
                scr_hbm.at[tok_v.at[l]], rows_v.at[buf], sem
            ).wait()

        def owin(l):
            return out_hbm.at[l, :, wid, :, :]

        def extract(l, buf, semw):
            for g in range(8):
                b0g = 16 * g

                def k_body(k, carry):
                    bvec = b0g + ((lane + k) & 15)
                    for je in range(4):
                        evec = evecs[je]
                        x = plsc.load_gather(rows_v.at[buf], [bvec, evec])
                        plsc.store_scatter(
                            ob_v.at[buf], [evec >> 3, evec & 7, bvec], x
                        )
                    return carry

                lax.fori_loop(0, 16, k_body, 0)
            return pltpu.async_copy(ob_v.at[buf], owin(l), semw)

        def drain_write(l, buf, semw):
            pltpu.make_async_copy(ob_v.at[buf], owin(l), semw).wait()

        fire(0, 0, semg0)

        def pair(j, carry):
            l0 = 2 * j
            fire(l0 + 1, 1, semg1)
            wait_gather(l0, 0, semg0)

            @pl.when(l0 >= 2)
            def _():
                drain_write(l0 - 2, 0, semw0)

            extract(l0, 0, semw0)

            @pl.when(l0 + 2 < _L)
            def _():
                fire(l0 + 2, 0, semg0)

            wait_gather(l0 + 1, 1, semg1)

            @pl.when(l0 >= 2)
            def _():
                drain_write(l0 - 1, 1, semw1)

            extract(l0 + 1, 1, semw1)
            return carry

        lax.fori_loop(0, _L // 2, pair, 0)
        drain_write(_L - 2, 0, semw0)
        drain_write(_L - 1, 1, semw1)

    return kb(tokens_t, scr)


def kernel(tokens, table):
    table_t = table.T
    tokens_t = tokens.T.astype(jnp.int32)
    scr = _transpose_table(table_t)
    scr_rows = scr.reshape(2 * _SCR_ROWS, _EMB)
    out5 = _gather_rows(tokens_t, scr_rows)
    return out5.transpose(2, 4, 0, 1, 3).reshape(_B, _L, _EMB)


# speedup vs baseline: 1.7902x; 1.0439x over previous
import functools

import jax
import jax.numpy as jnp
from jax import lax
from jax.experimental import pallas as pl
from jax.experimental.pallas import tpu as pltpu
from jax.experimental.pallas import tpu_sc as plsc

_VOCAB = 1000000
_EMB = 64
_B = 4096
_L = 200
_SCALE = 8.0

_NC = 2
_NS = 16
_NW = _NC * _NS

_W = 512
_NBLK = (_VOCAB + 127) // 128
_SCR_ROWS = (_NBLK * 128) // 2
_NBIG = (_NBLK * 128 + _W - 1) // _W
_V_LAST = _NBLK * 128 - _W
_BIG_LO = _NBIG // _NW
_BIG_EXTRA = _NBIG - _BIG_LO * _NW


def _transpose_table(table_t):
    mesh = plsc.VectorSubcoreMesh(core_axis_name="c", subcore_axis_name="s")

    @functools.partial(
        pl.kernel,
        mesh=mesh,
        compiler_params=pltpu.CompilerParams(needs_layout_passes=False),
        out_type=jax.ShapeDtypeStruct((_SCR_ROWS * 128,), jnp.float32),
        scratch_types=[
            pltpu.VMEM((2, _EMB, _W), jnp.float32),
            pltpu.VMEM((_EMB * 128,), jnp.float32),
            pltpu.VMEM((_EMB * 128,), jnp.float32),
            pltpu.VMEM((_EMB * 128,), jnp.float32),
            pltpu.VMEM((_EMB * 128,), jnp.float32),
            pltpu.SemaphoreType.DMA,
            pltpu.SemaphoreType.DMA,
            pltpu.SemaphoreType.DMA,
        ],
    )
    def ka(tab_hbm, scr_hbm, vin, vo0, vo1, vo2, vo3, semi0, semi1, semo):
        vouts = [vo0, vo1, vo2, vo3]
        wid = lax.axis_index("s") * _NC + lax.axis_index("c")
        nblk = jnp.where(wid < _BIG_EXTRA, _BIG_LO + 1, _BIG_LO)
        blk0 = wid * _BIG_LO + jnp.minimum(wid, _BIG_EXTRA)

        lane = lax.iota(jnp.int32, 16)
        evecs = [16 * je + lane for je in range(4)]

        def v0_of(i):
            return pl.multiple_of(
                jnp.minimum((blk0 + i) * _W, _V_LAST), 128
            )

        def vsrc(i):
            return tab_hbm.at[:, pl.ds(v0_of(i), _W)]

        def fire_in(i, buf, sem):
            return pltpu.async_copy(vsrc(i), vin.at[buf], sem)

        def wait_in(i, buf, sem):
            pltpu.make_async_copy(vsrc(i), vin.at[buf], sem).wait()

        def drain_out():
            pltpu.make_async_copy(
                vo0, scr_hbm.at[pl.ds(0, _EMB * 128)], semo
            ).wait()

        def transpose_block(i, buf, first):
            for k4 in range(4):
                @pl.when(jnp.logical_not(first))
                def _():
                    drain_out()

                for g in range(8):
                    v0g = 128 * k4 + 16 * g

                    def k_body(k, carry):
                        vloc = 16 * g + ((lane + k) & 15)
                        vvec = 128 * k4 + vloc
                        vvec64 = vloc * _EMB
                        for je in range(4):
                            evec = evecs[je]
                            x = plsc.load_gather(vin.at[buf], [evec, vvec])
                            plsc.store_scatter(
                                vouts[k4], [vvec64 + evec], x
                            )
                        return carry

                    lax.fori_loop(0, 16, k_body, 0)
                f0 = pl.multiple_of(
                    (v0_of(i) // 2 + 64 * k4) * 128, _EMB * 128
                )
                pltpu.async_copy(
                    vouts[k4], scr_hbm.at[pl.ds(f0, _EMB * 128)], semo
                )

        fire_in(0, 0, semi0)

        def pair(j, carry):
            i0 = 2 * j

            @pl.when(i0 + 1 < nblk)
            def _():
                fire_in(i0 + 1, 1, semi1)

            @pl.when(i0 < nblk)
            def _():
                wait_in(i0, 0, semi0)
                transpose_block(i0, 0, i0 == 0)

            @pl.when(i0 + 2 < nblk)
            def _():
                fire_in(i0 + 2, 0, semi0)

            @pl.when(i0 + 1 < nblk)
            def _():
                wait_in(i0 + 1, 1, semi1)
                transpose_block(i0 + 1, 1, False)

            return carry

        lax.fori_loop(0, (_BIG_LO + 2) // 2, pair, 0)
        for _ in range(4):
            drain_out()

    return ka(table_t)


def _gather_rows(tokens_t, scr):
    mesh = plsc.VectorSubcoreMesh(core_axis_name="c", subcore_axis_name="s")

    @functools.partial(
        pl.kernel,
        mesh=mesh,
        compiler_params=pltpu.CompilerParams(
            needs_layout_passes=False, use_tc_tiling_on_sc=False
        ),
        out_type=jax.ShapeDtypeStruct((_L, 8, _NW, 8, 128), jnp.float32),
        scratch_types=[
            pltpu.VMEM((_L, 128), jnp.int32),
            pltpu.VMEM((2, 128, _EMB), jnp.float32),
            pltpu.VMEM((2, 8, 8, 128), jnp.float32),
            pltpu.SemaphoreType.DMA,
            pltpu.SemaphoreType.DMA,
            pltpu.SemaphoreType.DMA,
            pltpu.SemaphoreType.DMA,
        ],
    )
    def kb(tok_hbm, scr_hbm, out_hbm, tok_v, rows_v, ob_v,
           semg0, semg1, semw0, semw1):
        wid = lax.axis_index("s") * _NC + lax.axis_index("c")
        b0 = pl.multiple_of(wid * 128, 128)
        lane = lax.iota(jnp.int32, 16)
        evecs = [16 * je + lane for je in range(4)]

        pltpu.sync_copy(tok_hbm.at[:, pl.ds(b0, 128)], tok_v)

        def fire(l, buf, sem):
            return pltpu.async_copy(
                scr_hbm.at[tok_v.at[l]], rows_v.at[buf], sem
            )

        def wait_gather(l, buf, sem):
            pltpu.make_async_copy(
                scr_hbm.at[tok_v.at[l]], rows_v.at[buf], sem
            ).wait()

        def owin(l):
            return out_hbm.at[l, :, wid, :, :]

        def extract(l, buf, semw):
            for g in range(8):
                b0g = 16 * g

                def k_body(k, carry):
                    bvec = b0g + ((lane + k) & 15)
                    for je in range(4):
                        evec = evecs[je]
                        x = plsc.load_gather(rows_v.at[buf], [bvec, evec])
                        plsc.store_scatter(
                            ob_v.at[buf],
                            [evec >> 3, evec & 7, bvec],
                            x * _SCALE,
                        )
                    return carry

                lax.fori_loop(0, 16, k_body, 0)
            return pltpu.async_copy(ob_v.at[buf], owin(l), semw)

        def drain_write(l, buf, semw):
            pltpu.make_async_copy(ob_v.at[buf], owin(l), semw).wait()

        fire(0, 0, semg0)

        def pair(j, carry):
            l0 = 2 * j
            fire(l0 + 1, 1, semg1)
            wait_gather(l0, 0, semg0)

            @pl.when(l0 >= 2)
            def _():
                drain_write(l0 - 2, 0, semw0)

            extract(l0, 0, semw0)

            @pl.when(l0 + 2 < _L)
            def _():
                fire(l0 + 2, 0, semg0)

            wait_gather(l0 + 1, 1, semg1)

            @pl.when(l0 >= 2)
            def _():
                drain_write(l0 - 1, 1, semw1)

            extract(l0 + 1, 1, semw1)
            return carry

        lax.fori_loop(0, _L // 2, pair, 0)
        drain_write(_L - 2, 0, semw0)
        drain_write(_L - 1, 1, semw1)

    return kb(tokens_t, scr)


def kernel(tokens, table):
    table_t = table.T
    tokens_t = tokens.T.astype(jnp.int32)
    scr = _transpose_table(table_t)
    scr_rows = scr.reshape(2 * _SCR_ROWS, _EMB)
    del scr
    out5 = _gather_rows(tokens_t, scr_rows)
    return out5.transpose(2, 4, 0, 1, 3).reshape(_B, _L, _EMB)


# speedup vs baseline: 2.5373x; 1.4174x over previous
import functools

import jax
import jax.numpy as jnp
from jax import lax
from jax.experimental import pallas as pl
from jax.experimental.pallas import tpu as pltpu
from jax.experimental.pallas import tpu_sc as plsc

_VOCAB = 1000000
_EMB = 64
_B = 4096
_L = 200
_SCALE = 8.0

_NC = 2
_NS = 16
_NW = _NC * _NS

_W = 512
_NBLK = (_VOCAB + 127) // 128
_SCR_ROWS = (_NBLK * 128) // 2
_NBIG = (_NBLK * 128 + _W - 1) // _W
_V_LAST = _NBLK * 128 - _W
_BIG_LO = _NBIG // _NW
_BIG_EXTRA = _NBIG - _BIG_LO * _NW


def _transpose_table(table_t):
    mesh = plsc.VectorSubcoreMesh(core_axis_name="c", subcore_axis_name="s")

    @functools.partial(
        pl.kernel,
        mesh=mesh,
        compiler_params=pltpu.CompilerParams(needs_layout_passes=False),
        out_type=jax.ShapeDtypeStruct((_SCR_ROWS * 128,), jnp.float32),
        scratch_types=[
            pltpu.VMEM((2, _EMB, _W), jnp.float32),
            pltpu.VMEM((_EMB * 128,), jnp.float32),
            pltpu.VMEM((_EMB * 128,), jnp.float32),
            pltpu.VMEM((_EMB * 128,), jnp.float32),
            pltpu.VMEM((_EMB * 128,), jnp.float32),
            pltpu.SemaphoreType.DMA,
            pltpu.SemaphoreType.DMA,
            pltpu.SemaphoreType.DMA,
        ],
    )
    def ka(tab_hbm, scr_hbm, vin, vo0, vo1, vo2, vo3, semi0, semi1, semo):
        vouts = [vo0, vo1, vo2, vo3]
        wid = lax.axis_index("s") * _NC + lax.axis_index("c")
        nblk = jnp.where(wid < _BIG_EXTRA, _BIG_LO + 1, _BIG_LO)
        blk0 = wid * _BIG_LO + jnp.minimum(wid, _BIG_EXTRA)

        lane = lax.iota(jnp.int32, 16)
        evecs = [16 * je + lane for je in range(4)]

        def v0_of(i):
            return pl.multiple_of(
                jnp.minimum((blk0 + i) * _W, _V_LAST), 128
            )

        def vsrc(i):
            return tab_hbm.at[:, pl.ds(v0_of(i), _W)]

        def fire_in(i, buf, sem):
            return pltpu.async_copy(vsrc(i), vin.at[buf], sem)

        def wait_in(i, buf, sem):
            pltpu.make_async_copy(vsrc(i), vin.at[buf], sem).wait()

        def drain_out():
            pltpu.make_async_copy(
                vo0, scr_hbm.at[pl.ds(0, _EMB * 128)], semo
            ).wait()

        def transpose_block(i, buf, first):
            for k4 in range(4):
                @pl.when(jnp.logical_not(first))
                def _():
                    drain_out()

                for g in range(8):
                    v0g = 128 * k4 + 16 * g

                    def k_body(k, carry):
                        vloc = 16 * g + ((lane + k) & 15)
                        vvec = 128 * k4 + vloc
                        vvec64 = vloc * _EMB
                        xs = [
                            plsc.load_gather(vin.at[buf], [evecs[je], vvec])
                            for je in range(4)
                        ]
                        for je in range(4):
                            plsc.store_scatter(
                                vouts[k4], [vvec64 + evecs[je]], xs[je]
                            )
                        return carry

                    lax.fori_loop(0, 16, k_body, 0)
                f0 = pl.multiple_of(
                    (v0_of(i) // 2 + 64 * k4) * 128, _EMB * 128
                )
                pltpu.async_copy(
                    vouts[k4], scr_hbm.at[pl.ds(f0, _EMB * 128)], semo
                )

        fire_in(0, 0, semi0)

        def pair(j, carry):
            i0 = 2 * j

            @pl.when(i0 + 1 < nblk)
            def _():
                fire_in(i0 + 1, 1, semi1)

            @pl.when(i0 < nblk)
            def _():
                wait_in(i0, 0, semi0)
                transpose_block(i0, 0, i0 == 0)

            @pl.when(i0 + 2 < nblk)
            def _():
                fire_in(i0 + 2, 0, semi0)

            @pl.when(i0 + 1 < nblk)
            def _():
                wait_in(i0 + 1, 1, semi1)
                transpose_block(i0 + 1, 1, False)

            return carry

        lax.fori_loop(0, (_BIG_LO + 2) // 2, pair, 0)
        for _ in range(4):
            drain_out()

    return ka(table_t)


def _gather_rows(tokens_t, scr):
    mesh = plsc.VectorSubcoreMesh(core_axis_name="c", subcore_axis_name="s")

    @functools.partial(
        pl.kernel,
        mesh=mesh,
        compiler_params=pltpu.CompilerParams(
            needs_layout_passes=False, use_tc_tiling_on_sc=False
        ),
        out_type=jax.ShapeDtypeStruct((_L, 8, _NW, 8, 128), jnp.float32),
        scratch_types=[
            pltpu.VMEM((_L, 128), jnp.int32),
            pltpu.VMEM((2, 128, _EMB), jnp.float32),
            pltpu.VMEM((2, 8, 8, 128), jnp.float32),
            pltpu.SemaphoreType.DMA,
            pltpu.SemaphoreType.DMA,
            pltpu.SemaphoreType.DMA,
            pltpu.SemaphoreType.DMA,
        ],
    )
    def kb(tok_hbm, scr_hbm, out_hbm, tok_v, rows_v, ob_v,
           semg0, semg1, semw0, semw1):
        wid = lax.axis_index("s") * _NC + lax.axis_index("c")
        b0 = pl.multiple_of(wid * 128, 128)
        lane = lax.iota(jnp.int32, 16)
        evecs = [16 * je + lane for je in range(4)]

        pltpu.sync_copy(tok_hbm.at[:, pl.ds(b0, 128)], tok_v)

        def fire(l, buf, sem):
            return pltpu.async_copy(
                scr_hbm.at[tok_v.at[l]], rows_v.at[buf], sem
            )

        def wait_gather(l, buf, sem):
            pltpu.make_async_copy(
                scr_hbm.at[tok_v.at[l]], rows_v.at[buf], sem
            ).wait()

        def owin(l):
            return out_hbm.at[l, :, wid, :, :]

        def extract(l, buf, semw):
            for g in range(8):
                b0g = 16 * g

                def k_body(k, carry):
                    bvec = b0g + ((lane + k) & 15)
                    for je in range(4):
                        evec = evecs[je]
                        x = plsc.load_gather(rows_v.at[buf], [bvec, evec])
                        plsc.store_scatter(
                            ob_v.at[buf],
                            [evec >> 3, evec & 7, bvec],
                            x * _SCALE,
                        )
                    return carry

                lax.fori_loop(0, 16, k_body, 0)
            return pltpu.async_copy(ob_v.at[buf], owin(l), semw)

        def drain_write(l, buf, semw):
            pltpu.make_async_copy(ob_v.at[buf], owin(l), semw).wait()

        fire(0, 0, semg0)

        def pair(j, carry):
            l0 = 2 * j
            fire(l0 + 1, 1, semg1)
            wait_gather(l0, 0, semg0)

            @pl.when(l0 >= 2)
            def _():
                drain_write(l0 - 2, 0, semw0)

            extract(l0, 0, semw0)

            @pl.when(l0 + 2 < _L)
            def _():
                fire(l0 + 2, 0, semg0)

            wait_gather(l0 + 1, 1, semg1)

            @pl.when(l0 >= 2)
            def _():
                drain_write(l0 - 1, 1, semw1)

            extract(l0 + 1, 1, semw1)
            return carry

        lax.fori_loop(0, _L // 2, pair, 0)
        drain_write(_L - 2, 0, semw0)
        drain_write(_L - 1, 1, semw1)

    return kb(tokens_t, scr)


def kernel(tokens, table):
    table_t = table.T
    tokens_t = tokens.T.astype(jnp.int32)
    scr = _transpose_table(table_t)
    scr_rows = scr.reshape(2 * _SCR_ROWS, _EMB)
    del scr
    out5 = _gather_rows(tokens_t, scr_rows)
    return out5.transpose(2, 4, 0, 1, 3).reshape(_B, _L, _EMB)


# speedup vs baseline: 4.3474x; 1.7134x over previous
import functools

import jax
import jax.numpy as jnp
from jax import lax
from jax.experimental import pallas as pl
from jax.experimental.pallas import tpu as pltpu
from jax.experimental.pallas import tpu_sc as plsc

_VOCAB = 1000000
_EMB = 64
_B = 4096
_L = 200
_SCALE = 8.0

_NC = 2
_NS = 16
_NW = _NC * _NS

_W = 512
_NBLK = (_VOCAB + 127) // 128
_SCR_ROWS = (_NBLK * 128) // 2
_NBIG = (_NBLK * 128 + _W - 1) // _W
_V_LAST = _NBLK * 128 - _W
_BIG_LO = _NBIG // _NW
_BIG_EXTRA = _NBIG - _BIG_LO * _NW


def _transpose_table(table_t):
    mesh = plsc.VectorSubcoreMesh(core_axis_name="c", subcore_axis_name="s")

    @functools.partial(
        pl.kernel,
        mesh=mesh,
        compiler_params=pltpu.CompilerParams(needs_layout_passes=False),
        out_type=jax.ShapeDtypeStruct((_SCR_ROWS * 128,), jnp.float32),
        scratch_types=[
            pltpu.VMEM((2, _EMB, _W), jnp.float32),
            pltpu.VMEM((_EMB * 128,), jnp.float32),
            pltpu.VMEM((_EMB * 128,), jnp.float32),
            pltpu.VMEM((_EMB * 128,), jnp.float32),
            pltpu.VMEM((_EMB * 128,), jnp.float32),
            pltpu.SemaphoreType.DMA,
            pltpu.SemaphoreType.DMA,
            pltpu.SemaphoreType.DMA,
        ],
    )
    def ka(tab_hbm, scr_hbm, vin, vo0, vo1, vo2, vo3, semi0, semi1, semo):
        vouts = [vo0, vo1, vo2, vo3]
        wid = lax.axis_index("s") * _NC + lax.axis_index("c")
        nblk = jnp.where(wid < _BIG_EXTRA, _BIG_LO + 1, _BIG_LO)
        blk0 = wid * _BIG_LO + jnp.minimum(wid, _BIG_EXTRA)

        lane = lax.iota(jnp.int32, 16)
        evecs = [16 * je + lane for je in range(4)]

        def v0_of(i):
            return pl.multiple_of(
                jnp.minimum((blk0 + i) * _W, _V_LAST), 128
            )

        def vsrc(i):
            return tab_hbm.at[:, pl.ds(v0_of(i), _W)]

        def fire_in(i, buf, sem):
            return pltpu.async_copy(vsrc(i), vin.at[buf], sem)

        def wait_in(i, buf, sem):
            pltpu.make_async_copy(vsrc(i), vin.at[buf], sem).wait()

        def drain_out():
            pltpu.make_async_copy(
                vo0, scr_hbm.at[pl.ds(0, _EMB * 128)], semo
            ).wait()

        def transpose_block(i, buf, first):
            for k4 in range(4):
                @pl.when(jnp.logical_not(first))
                def _():
                    drain_out()

                for g in range(8):
                    v0g = 128 * k4 + 16 * g

                    def k_body(k, carry):
                        vloc = 16 * g + ((lane + k) & 15)
                        vvec = 128 * k4 + vloc
                        vvec64 = vloc * _EMB
                        xs = [
                            plsc.load_gather(vin.at[buf], [evecs[je], vvec])
                            for je in range(4)
                        ]
                        for je in range(4):
                            plsc.store_scatter(
                                vouts[k4], [vvec64 + evecs[je]], xs[je]
                            )
                        return carry

                    lax.fori_loop(0, 16, k_body, 0)
                f0 = pl.multiple_of(
                    (v0_of(i) // 2 + 64 * k4) * 128, _EMB * 128
                )
                pltpu.async_copy(
                    vouts[k4], scr_hbm.at[pl.ds(f0, _EMB * 128)], semo
                )

        fire_in(0, 0, semi0)

        def pair(j, carry):
            i0 = 2 * j

            @pl.when(i0 + 1 < nblk)
            def _():
                fire_in(i0 + 1, 1, semi1)

            @pl.when(i0 < nblk)
            def _():
                wait_in(i0, 0, semi0)
                transpose_block(i0, 0, i0 == 0)

            @pl.when(i0 + 2 < nblk)
            def _():
                fire_in(i0 + 2, 0, semi0)

            @pl.when(i0 + 1 < nblk)
            def _():
                wait_in(i0 + 1, 1, semi1)
                transpose_block(i0 + 1, 1, False)

            return carry

        lax.fori_loop(0, (_BIG_LO + 2) // 2, pair, 0)
        for _ in range(4):
            drain_out()

    return ka(table_t)


def _gather_rows(tokens_t, scr):
    mesh = plsc.VectorSubcoreMesh(core_axis_name="c", subcore_axis_name="s")

    @functools.partial(
        pl.kernel,
        mesh=mesh,
        compiler_params=pltpu.CompilerParams(
            needs_layout_passes=False, use_tc_tiling_on_sc=False
        ),
        out_type=jax.ShapeDtypeStruct((_L, 8, _NW, 8, 128), jnp.float32),
        scratch_types=[
            pltpu.VMEM((_L, 128), jnp.int32),
            pltpu.VMEM((2, 128, _EMB), jnp.float32),
            pltpu.VMEM((2, 8, 8, 128), jnp.float32),
            pltpu.SemaphoreType.DMA,
            pltpu.SemaphoreType.DMA,
            pltpu.SemaphoreType.DMA,
            pltpu.SemaphoreType.DMA,
        ],
    )
    def kb(tok_hbm, scr_hbm, out_hbm, tok_v, rows_v, ob_v,
           semg0, semg1, semw0, semw1):
        wid = lax.axis_index("s") * _NC + lax.axis_index("c")
        b0 = pl.multiple_of(wid * 128, 128)
        lane = lax.iota(jnp.int32, 16)
        evecs = [16 * je + lane for je in range(4)]

        pltpu.sync_copy(tok_hbm.at[:, pl.ds(b0, 128)], tok_v)

        def fire(l, buf, sem):
            return pltpu.async_copy(
                scr_hbm.at[tok_v.at[l]], rows_v.at[buf], sem
            )

        def wait_gather(l, buf, sem):
            pltpu.make_async_copy(
                scr_hbm.at[tok_v.at[l]], rows_v.at[buf], sem
            ).wait()

        def owin(l):
            return out_hbm.at[l, :, wid, :, :]

        def extract(l, buf, semw):
            for g in range(8):
                b0g = 16 * g

                def k_body(k, carry):
                    bvec = b0g + ((lane + k) & 15)
                    xs = [
                        plsc.load_gather(rows_v.at[buf], [bvec, evecs[je]])
                        for je in range(4)
                    ]
                    for je in range(4):
                        evec = evecs[je]
                        plsc.store_scatter(
                            ob_v.at[buf],
                            [evec >> 3, evec & 7, bvec],
                            xs[je] * _SCALE,
                        )
                    return carry

                lax.fori_loop(0, 16, k_body, 0)
            return pltpu.async_copy(ob_v.at[buf], owin(l), semw)

        def drain_write(l, buf, semw):
            pltpu.make_async_copy(ob_v.at[buf], owin(l), semw).wait()

        fire(0, 0, semg0)

        def pair(j, carry):
            l0 = 2 * j
            fire(l0 + 1, 1, semg1)
            wait_gather(l0, 0, semg0)

            @pl.when(l0 >= 2)
            def _():
                drain_write(l0 - 2, 0, semw0)

            extract(l0, 0, semw0)

            @pl.when(l0 + 2 < _L)
            def _():
                fire(l0 + 2, 0, semg0)

            wait_gather(l0 + 1, 1, semg1)

            @pl.when(l0 >= 2)
            def _():
                drain_write(l0 - 1, 1, semw1)

            extract(l0 + 1, 1, semw1)
            return carry

        lax.fori_loop(0, _L // 2, pair, 0)
        drain_write(_L - 2, 0, semw0)
        drain_write(_L - 1, 1, semw1)

    return kb(tokens_t, scr)


def kernel(tokens, table):
    table_t = table.T
    tokens_t = tokens.T.astype(jnp.int32)
    scr = _transpose_table(table_t)
    scr_rows = scr.reshape(2 * _SCR_ROWS, _EMB)
    del scr
    out5 = _gather_rows(tokens_t, scr_rows)
    return out5.transpose(2, 4, 0, 1, 3).reshape(_B, _L, _EMB)
